# baseline (speedup 1.0000x reference)
import jax
import jax.numpy as jnp
from jax import lax
from jax.experimental import pallas as pl
from jax.experimental.pallas import tpu as pltpu
from jax.experimental.pallas import tpu_sc as plsc

B = 16384
V = 1000000
R = 1000
D = 64

NC = 2
NS = 16
L = 16
NW = NC * NS
BPW = B // NW
CHUNK = 128
NCHUNK = BPW // CHUNK
GPC = CHUNK // L

VB = 512
NBAND = V // VB
VTAIL = V - NBAND * VB
NI = 2 * B
JUNK = NI
NPAD = NI + 256
SLOTS = 128
IC = 128


def _rsqrt(x):
    i = plsc.bitcast(x, jnp.int32)
    y = plsc.bitcast(jnp.int32(0x5F3759DF) - (i >> 1), jnp.float32)
    xh = x * 0.5
    for _ in range(3):
        y = y * (1.5 - (xh * y) * y)
    return y


def _scalar_at(ref, j):
    vec = ref[pl.ds((j // L) * L, L)]
    lane = lax.iota(jnp.int32, L) == (j % L)
    return lax.reduce_max(jnp.where(lane, vec, jnp.int32(-2147483648)), (0,))


def _scatter_body(ent_hbm, sv_hbm, sb_hbm, lo_hbm, hi_hbm, scr_hbm,
                  slab_a, slab_b, slab_s, svbuf, sbbuf, sidx, rowbuf,
                  vlo, vhi, sem_a, sem_b, sem_c):
    wid = lax.axis_index("s") * NC + lax.axis_index("c")

    pltpu.sync_copy(lo_hbm.at[wid], vlo)
    pltpu.sync_copy(hi_hbm.at[wid], vhi)
    cnt = 61 + jnp.where(wid == 0, 1, 0)

    def process(slab, width, base, lo, hi):
        n_tot = hi - lo

        def ic_body(ic, carry):
            lo_ic = lo + ic * IC
            lo8 = (lo_ic // 8) * 8
            off = lo_ic - lo8
            m = jnp.minimum(IC, n_tot - ic * IC)
            pltpu.sync_copy(sv_hbm.at[pl.ds(lo8, IC + 8)], svbuf)
            pltpu.sync_copy(sb_hbm.at[pl.ds(lo8, IC + 8)], sbbuf)
            for s in range(2):
                for q in range(8):
                    sidx[s, pl.ds(q * L, L)] = jnp.full((L,), JUNK, jnp.int32)

            def k_body(k16, kc):
                kvec = k16 * L + lax.iota(jnp.int32, L)
                vvec = svbuf[pl.ds(k16 * L, L)]
                bvec = sbbuf[pl.ds(k16 * L, L)]
                valid = (kvec >= off) & (kvec < off + m)
                beff = jnp.where(valid, bvec, JUNK)
                vloc = jnp.clip(vvec - base, 0, width - 1)
                plsc.store_scatter(sidx, [kvec >> 7, kvec & 127], beff)
                for j in range(D):
                    cj = jnp.full((L,), j, jnp.int32)
                    val = plsc.load_gather(slab, [cj, vloc])
                    plsc.store_scatter(rowbuf, [kvec, cj], val)
                return kc

            lax.fori_loop(0, (off + m + L - 1) // L, k_body, 0)

            def s_body(s, sc):
                pltpu.async_copy(rowbuf.at[pl.ds(s * CHUNK, CHUNK)],
                                 scr_hbm.at[sidx.at[s]], sem_c).wait()
                return sc

            lax.fori_loop(0, (off + m + CHUNK - 1) // CHUNK, s_body, 0)
            return carry

        lax.fori_loop(0, (n_tot + IC - 1) // IC, ic_body, 0)

    def dochunk(j, slab_cur, slab_nxt, sem_cur, sem_nxt):
        g = wid + NW * j
        base = g * VB
        pltpu.make_async_copy(
            ent_hbm.at[:, pl.ds(base, VB)], slab_cur, sem_cur).wait()

        @pl.when(j + 1 < cnt)
        def _():
            pltpu.async_copy(
                ent_hbm.at[:, pl.ds((g + NW) * VB, VB)], slab_nxt, sem_nxt)

        process(slab_cur, VB, base, _scalar_at(vlo, j), _scalar_at(vhi, j))

    pltpu.async_copy(ent_hbm.at[:, pl.ds(wid * VB, VB)], slab_a, sem_a)

    def band_pair(i, carry):
        j0 = 2 * i

        @pl.when(j0 < cnt)
        def _():
            dochunk(j0, slab_a, slab_b, sem_a, sem_b)

        @pl.when(j0 + 1 < cnt)
        def _():
            dochunk(j0 + 1, slab_b, slab_a, sem_b, sem_a)

        return carry

    lax.fori_loop(0, 31, band_pair, 0)

    @pl.when(wid == NW - 1)
    def _():
        pltpu.async_copy(
            ent_hbm.at[:, pl.ds(NBAND * VB, VTAIL)], slab_s, sem_a).wait()
        process(slab_s, VTAIL, NBAND * VB,
                _scalar_at(vlo, 62), _scalar_at(vhi, 62))


def _distance_body(scr_hbm, li2_hbm, lp_hbm, rel_hbm, out_hbm,
                   idx_l, par_l, hslab, tslab, rbuf, outv, sem):
    wid = lax.axis_index("s") * NC + lax.axis_index("c")

    pltpu.sync_copy(li2_hbm.at[wid], idx_l)
    pltpu.sync_copy(lp_hbm.at[wid], par_l)

    for c in range(NCHUNK):
        rbase = wid * BPW + c * CHUNK
        descs = [
            pltpu.async_copy(scr_hbm.at[pl.ds(rbase, CHUNK)], hslab, sem),
            pltpu.async_copy(scr_hbm.at[pl.ds(B + rbase, CHUNK)], tslab, sem),
            pltpu.async_copy(rel_hbm.at[idx_l.at[c]], rbuf, sem),
        ]
        for d in descs:
            d.wait()

        def group(g, carry, c=c):
            row = g * L + lax.iota(jnp.int32, L)
            prl = par_l[c, pl.ds(g * L, L)]
            zero = jnp.zeros((L,), jnp.float32)
            hh = zero; tt = zero; rr = zero
            hr = zero; ht = zero; rt = zero
            for j in range(D):
                cj = jnp.full((L,), j, jnp.int32)
                h = plsc.load_gather(hslab, [row, cj])
                t = plsc.load_gather(tslab, [row, cj])
                r = plsc.load_gather(rbuf, [row, prl + j])
                hh = hh + h * h
                tt = tt + t * t
                rr = rr + r * r
                hr = hr + h * r
                ht = ht + h * t
                rt = rt + t * r
            a = 1.0 / jnp.maximum(hh * _rsqrt(hh), 1e-12)
            b = 1.0 / jnp.maximum(tt * _rsqrt(tt), 1e-12)
            d2 = ((a * hh) * a + rr + (b * tt) * b
                  + 2.0 * (a * hr) - 2.0 * ((a * ht) * b) - 2.0 * (b * rt))
            d2 = jnp.maximum(d2, 0.0)
            plsc.store_scatter(outv, [c * CHUNK + row], d2 * _rsqrt(d2))
            return carry

        lax.fori_loop(0, GPC, group, 0)

    pltpu.sync_copy(outv, out_hbm.at[pl.ds(wid * BPW, BPW)])


@jax.jit
def kernel(head_ind, label, tail_ind, ent_embs, rel_embs):
    mesh = plsc.VectorSubcoreMesh(core_axis_name="c", subcore_axis_name="s")
    cp = pltpu.CompilerParams(needs_layout_passes=False)

    run_scatter = pl.kernel(
        _scatter_body,
        mesh=mesh,
        compiler_params=cp,
        out_type=jax.ShapeDtypeStruct((NI + CHUNK, 2 * D), jnp.float32),
        scratch_types=[
            pltpu.VMEM((D, VB), jnp.float32),
            pltpu.VMEM((D, VB), jnp.float32),
            pltpu.VMEM((D, VTAIL), jnp.float32),
            pltpu.VMEM((IC + 8,), jnp.int32),
            pltpu.VMEM((IC + 8,), jnp.int32),
            pltpu.VMEM((2, CHUNK), jnp.int32),
            pltpu.VMEM((2 * CHUNK, 2 * D), jnp.float32),
            pltpu.VMEM((SLOTS,), jnp.int32),
            pltpu.VMEM((SLOTS,), jnp.int32),
            pltpu.SemaphoreType.DMA,
            pltpu.SemaphoreType.DMA,
            pltpu.SemaphoreType.DMA,
        ],
    )

    run_distance = pl.kernel(
        _distance_body,
        mesh=mesh,
        compiler_params=cp,
        out_type=jax.ShapeDtypeStruct((B,), jnp.float32),
        scratch_types=[
            pltpu.VMEM((NCHUNK, CHUNK), jnp.int32),
            pltpu.VMEM((NCHUNK, CHUNK), jnp.int32),
            pltpu.VMEM((CHUNK, 2 * D), jnp.float32),
            pltpu.VMEM((CHUNK, 2 * D), jnp.float32),
            pltpu.VMEM((CHUNK, 2 * D), jnp.float32),
            pltpu.VMEM((BPW,), jnp.float32),
            pltpu.SemaphoreType.DMA,
        ],
    )

    hi32 = head_ind.astype(jnp.int32)
    ti32 = tail_ind.astype(jnp.int32)
    li32 = label.astype(jnp.int32)

    v_all = jnp.concatenate([hi32, ti32])
    b_all = jnp.arange(NI, dtype=jnp.int32)
    sv, sb = lax.sort_key_val(v_all, b_all)
    sv_p = jnp.concatenate([sv, jnp.full((NPAD - NI,), V, jnp.int32)])
    sb_p = jnp.concatenate([sb, jnp.full((NPAD - NI,), JUNK, jnp.int32)])

    edges = jnp.arange(NBAND + 1, dtype=jnp.int32) * VB
    bnd = jnp.searchsorted(sv, edges).astype(jnp.int32)
    gm = jnp.minimum(
        jnp.arange(NW, dtype=jnp.int32)[:, None]
        + NW * jnp.arange(62, dtype=jnp.int32)[None, :],
        NBAND - 1)
    lo_tab = jnp.concatenate(
        [bnd[gm], jnp.broadcast_to(bnd[NBAND], (NW, 1)),
         jnp.zeros((NW, SLOTS - 63), jnp.int32)], axis=1)
    hi_tab = jnp.concatenate(
        [bnd[gm + 1], jnp.full((NW, 1), NI, jnp.int32),
         jnp.zeros((NW, SLOTS - 63), jnp.int32)], axis=1)

    ent_t = jnp.swapaxes(ent_embs, 0, 1)
    scr = run_scatter(ent_t, sv_p, sb_p, lo_tab, hi_tab)

    li2 = (li32 >> 1).reshape(NW, NCHUNK, CHUNK)
    lp = ((li32 & 1) << 6).reshape(NW, NCHUNK, CHUNK)
    r2 = rel_embs.reshape(R // 2, 2 * D)
    return run_distance(scr, li2, lp, r2)


# speedup vs baseline: 28.3779x; 28.3779x over previous
import jax
import jax.numpy as jnp
from jax import lax
from jax.experimental import pallas as pl
from jax.experimental.pallas import tpu as pltpu
from jax.experimental.pallas import tpu_sc as plsc

B = 16384
V = 1000000
R = 1000
D = 64

NC = 2
NS = 16
L = 16
NW = NC * NS
BPW = B // NW
CHUNK = 128
NCHUNK = BPW // CHUNK
GPC = CHUNK // L
DP = 128


def _rsqrt(x):
    i = plsc.bitcast(x, jnp.int32)
    y = plsc.bitcast(jnp.int32(0x5F3759DF) - (i >> 1), jnp.float32)
    xh = x * 0.5
    for _ in range(3):
        y = y * (1.5 - (xh * y) * y)
    return y


def _trans_e_body(hi_hbm, li_hbm, ti_hbm, ent_hbm, rel_hbm, out_hbm,
                  idx_h, idx_l, idx_t, hbuf, tbuf, rbuf, outv, sem):
    wid = lax.axis_index("s") * NC + lax.axis_index("c")

    pltpu.sync_copy(hi_hbm.at[wid], idx_h)
    pltpu.sync_copy(li_hbm.at[wid], idx_l)
    pltpu.sync_copy(ti_hbm.at[wid], idx_t)

    for c in range(NCHUNK):
        descs = [
            pltpu.async_copy(ent_hbm.at[idx_h.at[c]], hbuf, sem),
            pltpu.async_copy(ent_hbm.at[idx_t.at[c]], tbuf, sem),
            pltpu.async_copy(rel_hbm.at[idx_l.at[c]], rbuf, sem),
        ]
        for d in descs:
            d.wait()

        def group(g, carry, c=c):
            row = g * L + lax.iota(jnp.int32, L)
            zero = jnp.zeros((L,), jnp.float32)
            hh = zero; tt = zero; rr = zero
            hr = zero; ht = zero; rt = zero
            for j in range(D):
                col = jnp.full((L,), j, jnp.int32)
                h = plsc.load_gather(hbuf, [row, col])
                t = plsc.load_gather(tbuf, [row, col])
                r = plsc.load_gather(rbuf, [row, col])
                hh = hh + h * h
                tt = tt + t * t
                rr = rr + r * r
                hr = hr + h * r
                ht = ht + h * t
                rt = rt + t * r
            a = 1.0 / jnp.maximum(hh * _rsqrt(hh), 1e-12)
            b = 1.0 / jnp.maximum(tt * _rsqrt(tt), 1e-12)
            d2 = ((a * hh) * a + rr + (b * tt) * b
                  + 2.0 * (a * hr) - 2.0 * ((a * ht) * b) - 2.0 * (b * rt))
            d2 = jnp.maximum(d2, 0.0)
            plsc.store_scatter(outv, [c * CHUNK + row], d2 * _rsqrt(d2))
            return carry

        lax.fori_loop(0, GPC, group, 0)

    pltpu.sync_copy(outv, out_hbm.at[pl.ds(wid * BPW, BPW)])


@jax.jit
def kernel(head_ind, label, tail_ind, ent_embs, rel_embs):
    mesh = plsc.VectorSubcoreMesh(core_axis_name="c", subcore_axis_name="s")
    run = pl.kernel(
        _trans_e_body,
        mesh=mesh,
        compiler_params=pltpu.CompilerParams(needs_layout_passes=False),
        out_type=jax.ShapeDtypeStruct((B,), jnp.float32),
        scratch_types=[
            pltpu.VMEM((NCHUNK, CHUNK), jnp.int32),
            pltpu.VMEM((NCHUNK, CHUNK), jnp.int32),
            pltpu.VMEM((NCHUNK, CHUNK), jnp.int32),
            pltpu.VMEM((CHUNK, DP), jnp.float32),
            pltpu.VMEM((CHUNK, DP), jnp.float32),
            pltpu.VMEM((CHUNK, DP), jnp.float32),
            pltpu.VMEM((BPW,), jnp.float32),
            pltpu.SemaphoreType.DMA,
        ],
    )
    hi = head_ind.astype(jnp.int32).reshape(NW, NCHUNK, CHUNK)
    li = label.astype(jnp.int32).reshape(NW, NCHUNK, CHUNK)
    ti = tail_ind.astype(jnp.int32).reshape(NW, NCHUNK, CHUNK)
    eye = jnp.eye(D, DP, dtype=jnp.float32)
    ep = ent_embs @ eye
    rp = rel_embs @ eye
    return run(hi, li, ti, ep, rp)
